# baseline (speedup 1.0000x reference)
import functools

import jax
import jax.numpy as jnp
from jax import lax
from jax.experimental import pallas as pl
from jax.experimental.pallas import tpu as pltpu
from jax.experimental.pallas import tpu_sc as plsc

VOCAB_ = 1000000
DIM = 64
BATCH = 16384

_NC, _NS = 2, 16
_NW = _NC * _NS
_NBLK = (VOCAB_ + 127) // 128
_OWN = (_NBLK + _NW - 1) // _NW
_NSLOT = 3
_TRASH = BATCH
_EMAX = BATCH + 16


@functools.partial(
    pl.kernel,
    out_type=jax.ShapeDtypeStruct((BATCH + 8, 128), jnp.float32),
    mesh=plsc.VectorSubcoreMesh(core_axis_name="c", subcore_axis_name="s"),
    scratch_types=[
        pltpu.VMEM((_EMAX,), jnp.int32),
        pltpu.VMEM((_EMAX,), jnp.int32),
        pltpu.VMEM((_EMAX,), jnp.int32),
        pltpu.VMEM((_EMAX,), jnp.int32),
        pltpu.VMEM((272,), jnp.int32),
        pltpu.VMEM((288,), jnp.int32),
        pltpu.VMEM((32,), jnp.int32),
        pltpu.VMEM((2, _NSLOT, 64, 128), jnp.float32),
        pltpu.VMEM((64, 128), jnp.float32),
        pltpu.VMEM((8, 64), jnp.int32),
    ]
    + [pltpu.SemaphoreType.DMA] * (2 * _NSLOT),
    compiler_params=pltpu.CompilerParams(needs_layout_passes=False),
)
def _gather_d(table_t, nodes_hbm, outp, nodes_a, nn2, ii, nn, ublk, starts,
              bounce, blocks_v, staging_v, idxrow, *sems):
    wid = lax.axis_index("s") * _NC + lax.axis_index("c")
    lo = wid * _OWN
    hi = lo + _OWN
    iota16 = lax.iota(jnp.int32, 16)
    lane0 = iota16 == 0
    pltpu.sync_copy(nodes_hbm, nodes_a.at[pl.ds(0, BATCH)])

    def popcnt(m):
        return plsc.all_reduce_population_count(m)[0]

    def scan_body(t, cnt):
        vn = nodes_a[pl.ds(t * 16, 16)]
        b = vn >> 7
        m = (b >= lo) & (b < hi)
        vi = iota16 + t * 16
        plsc.store_compressed(ii.at[pl.ds(cnt, 16)], vi, mask=m)
        plsc.store_compressed(nn.at[pl.ds(cnt, 16)], vn, mask=m)
        return cnt + popcnt(m)

    e_cnt = lax.fori_loop(0, BATCH // 16, scan_body, 0)
    n_t16 = (e_cnt + 15) >> 4

    def radix_pass(src_i, src_n, dst_i, dst_n, shift):
        cnt2 = 0
        for d in range(16):
            def body(t, cnt, d=d):
                vn = src_n[pl.ds(t * 16, 16)]
                vi = src_i[pl.ds(t * 16, 16)]
                dig = (((vn >> 7) - lo) >> shift) & 15
                valid = (t * 16 + iota16) < e_cnt
                m = (dig == d) & valid
                plsc.store_compressed(dst_n.at[pl.ds(cnt, 16)], vn, mask=m)
                plsc.store_compressed(dst_i.at[pl.ds(cnt, 16)], vi, mask=m)
                return cnt + popcnt(m)

            cnt2 = lax.fori_loop(0, n_t16, body, cnt2)

    radix_pass(ii, nn, nodes_a, nn2, 0)
    radix_pass(nodes_a, nn2, ii, nn, 4)

    for z in range(272 // 16):
        ublk[pl.ds(z * 16, 16)] = jnp.zeros((16,), jnp.int32)
    for z in range(288 // 16):
        starts[pl.ds(z * 16, 16)] = jnp.full((16,), e_cnt, dtype=jnp.int32)

    def uniq_body(t, carry):
        nu, prevb = carry
        vn = nn[pl.ds(t * 16, 16)]
        b = vn >> 7
        plsc.store_compressed(bounce.at[pl.ds(0, 16)],
                              jnp.full((16,), prevb, dtype=jnp.int32),
                              mask=lane0)
        bounce[pl.ds(1, 16)] = b
        sh = bounce[pl.ds(0, 16)]
        valid = (t * 16 + iota16) < e_cnt
        newf = (b != sh) & valid
        plsc.store_compressed(ublk.at[pl.ds(nu, 16)], b, mask=newf)
        plsc.store_compressed(starts.at[pl.ds(nu, 16)], iota16 + t * 16,
                              mask=newf)
        return nu + popcnt(newf), b[15]

    nu, _ = lax.fori_loop(0, n_t16, uniq_body, (0, -1))
    plsc.store_compressed(starts.at[pl.ds(nu, 16)],
                          jnp.full((16,), e_cnt, dtype=jnp.int32), mask=lane0)
    for z in range(4):
        idxrow[0, pl.ds(z * 16, 16)] = jnp.full((16,), _TRASH, dtype=jnp.int32)

    def fetch_sb(s, par):
        for r in range(_NSLOT):
            k = jnp.minimum(s * _NSLOT + r, jnp.maximum(nu - 1, 0))
            bid = ublk[pl.ds(k, 16)][0]
            col = pl.multiple_of(bid * 128, 128)
            pltpu.async_copy(
                table_t.at[pl.ds(0, 64), pl.ds(col, 128)],
                blocks_v.at[par, r],
                sems[par * _NSLOT + r],
            )

    def drain_sb(par):
        for r in range(_NSLOT):
            pltpu.make_async_copy(
                table_t.at[pl.ds(0, 64), pl.ds(0, 128)],
                blocks_v.at[par, r],
                sems[par * _NSLOT + r],
            ).wait()

    def extract_sb(s, par):
        for r in range(_NSLOT):
            k = s * _NSLOT + r
            s0 = starts[pl.ds(k, 16)][0]
            s1 = starts[pl.ds(k + 1, 16)][0]

            def ent(e, _, r=r, par=par):
                vn = nn[pl.ds(e, 16)][0]
                vi = ii[pl.ds(e, 16)][0]
                c = jnp.full((16,), vn & 127, dtype=jnp.int32)
                slot = e & 63
                for q in range(4):
                    v = plsc.load_gather(blocks_v.at[par, r],
                                         [iota16 + q * 16, c])
                    staging_v[slot, pl.ds(q * 16, 16)] = v
                plsc.store_scatter(
                    idxrow,
                    [jnp.zeros((16,), jnp.int32),
                     jnp.full((16,), slot, dtype=jnp.int32)],
                    jnp.full((16,), vi, dtype=jnp.int32),
                    mask=lane0,
                )

                def flush():
                    pltpu.sync_copy(staging_v, outp.at[idxrow.at[0]])
                    for z in range(4):
                        idxrow[0, pl.ds(z * 16, 16)] = jnp.full(
                            (16,), _TRASH, dtype=jnp.int32)
                    return 0

                lax.cond(slot == 63, flush, lambda: 0)
                return 0

            lax.fori_loop(s0, s1, ent, 0)

    n_sb = (nu + _NSLOT - 1) // _NSLOT
    n_pair = (n_sb + 1) >> 1
    fetch_sb(0, 0)

    def pair_body(p, _):
        s = p * 2
        fetch_sb(s + 1, 1)
        drain_sb(0)
        extract_sb(s, 0)
        fetch_sb(s + 2, 0)
        drain_sb(1)
        extract_sb(s + 1, 1)
        return 0

    lax.fori_loop(0, n_pair, pair_body, 0)
    drain_sb(0)
    extract_sb(2 * n_pair, 0)

    def final_flush():
        pltpu.sync_copy(staging_v, outp.at[idxrow.at[0]])
        return 0

    lax.cond((e_cnt & 63) != 0, final_flush, lambda: 0)


def kernel(features, nodes):
    outp = _gather_d(features.T, nodes)
    return outp[:BATCH, :DIM]
